# baseline (speedup 1.0000x reference)
import jax
import jax.numpy as jnp
from jax import lax
from jax.experimental import pallas as pl
from jax.experimental.pallas import tpu as pltpu
from jax.experimental.pallas import tpu_sc as plsc

N = 10000
E = 320000
D = 128
R = 8
RN = R * N
NC = 2
NS = 16
L = 16
NW = NC * NS
C = 128
G = E // C
GBASE = G // NW
GEXTRA = G - GBASE * NW
CBASE = G // NS
CEXTRA = G - CBASE * NS
RSZ = 3 * C
NB = 3
NBM = 2
NP = 10240
RPT = NP // NS
CPT = 5120
CNTP = NS * CPT
PIECE = 640

_MESH = plsc.VectorSubcoreMesh(core_axis_name="c", subcore_axis_name="s")


def _sc_count_body(meta_hbm, zeros_hbm, counts_hbm,
                   mb, idxb, onesb, cacc, msem, ssem):
    c = lax.axis_index("c")
    s = lax.axis_index("s")
    wid = c * NS + s
    nch = GBASE + jnp.where(wid < GEXTRA, 1, 0)
    pltpu.sync_copy(zeros_hbm.at[pl.ds(s * CPT, CPT)],
                    cacc.at[pl.ds(s * CPT, CPT)])
    for j in range(C // L):
        onesb[pl.ds(j * L, L)] = jnp.full((L,), 1.0, jnp.float32)
    plsc.subcore_barrier()

    def fire_meta(k):
        g = k * NW + wid
        b = k % NB
        pltpu.async_copy(meta_hbm.at[pl.ds(g * RSZ, RSZ)], mb.at[b],
                         msem.at[b])

    def wait_meta(b):
        pltpu.make_async_copy(meta_hbm.at[pl.ds(0, RSZ)], mb.at[b],
                              msem.at[b]).wait()

    fire_meta(0)
    fire_meta(1)

    def step(k, carry):
        b = lax.rem(k, NB)
        wait_meta(b)

        @pl.when(k + 2 < nch)
        def _():
            fire_meta(k + 2)

        @pl.when(k >= NB)
        def _():
            pltpu.make_async_copy(onesb, cacc.at[idxb.at[b]],
                                  ssem.at[b]).wait()

        for j in range(C // L):
            ty = mb[b, pl.ds(C + j * L, L)]
            dv = mb[b, pl.ds(2 * C + j * L, L)]
            idxb[b, pl.ds(j * L, L)] = ty * N + dv
        pltpu.async_copy(onesb, cacc.at[idxb.at[b]], ssem.at[b], add=True)
        return carry

    lax.fori_loop(0, nch, step, 0)
    for b in range(NB):
        pltpu.make_async_copy(onesb, cacc.at[idxb.at[b]], ssem.at[b]).wait()
    plsc.subcore_barrier()
    pltpu.sync_copy(cacc.at[pl.ds(s * CPT, CPT)],
                    counts_hbm.at[pl.ds(c * CNTP + s * CPT, CPT)])


_sc_count = pl.kernel(
    _sc_count_body,
    mesh=_MESH,
    out_type=jax.ShapeDtypeStruct((NC * CNTP,), jnp.float32),
    scratch_types=[
        pltpu.VMEM((NB, RSZ), jnp.int32),
        pltpu.VMEM((NB, C), jnp.int32),
        pltpu.VMEM((C,), jnp.float32),
        pltpu.VMEM_SHARED((CNTP,), jnp.float32),
        pltpu.SemaphoreType.DMA((NB,)),
        pltpu.SemaphoreType.DMA((NB,)),
    ],
)


def _sc_main_body(meta_hbm, xr_hbm, counts_hbm, base_hbm, zeros_hbm,
                  msg_hbm, inv_hbm, mb, idxb, scaleb, rowsb, cbuf, cbuf2,
                  acc, msem, gsem, ssem):
    c = lax.axis_index("c")
    s = lax.axis_index("s")
    wid = c * NS + s
    nch = GBASE + jnp.where(wid < GEXTRA, 1, 0)

    @pl.when(c == 0)
    def _():
        pltpu.sync_copy(base_hbm.at[pl.ds(s * RPT, RPT)],
                        acc.at[pl.ds(s * RPT, RPT)])

    @pl.when(c == 1)
    def _():
        pltpu.sync_copy(zeros_hbm.at[pl.ds(s * RPT, RPT)],
                        acc.at[pl.ds(s * RPT, RPT)])

    pltpu.sync_copy(counts_hbm.at[pl.ds(c * CNTP + s * CPT, CPT)], cbuf)
    oc = 1 - c
    for p in range(CPT // PIECE):
        pltpu.sync_copy(
            counts_hbm.at[pl.ds(oc * CNTP + s * CPT + p * PIECE, PIECE)],
            cbuf2)

        @plsc.parallel_loop(0, PIECE // L, 1, unroll=4)
        def _inv(i):
            v = cbuf[pl.ds(p * PIECE + i * L, L)] + cbuf2[pl.ds(i * L, L)]
            cbuf[pl.ds(p * PIECE + i * L, L)] = 1.0 / jnp.maximum(v, 1.0)

    pltpu.sync_copy(cbuf, inv_hbm.at[pl.ds(c * CNTP + s * CPT, CPT)])
    plsc.subcore_barrier()

    def fire_meta(k):
        g = k * NW + wid
        b = k % NBM
        pltpu.async_copy(meta_hbm.at[pl.ds(g * RSZ, RSZ)], mb.at[b],
                         msem.at[b])

    def wait_meta(b):
        pltpu.make_async_copy(meta_hbm.at[pl.ds(0, RSZ)], mb.at[b],
                              msem.at[b]).wait()

    coff = c * CNTP

    def compute_idx(b):
        for j in range(C // L):
            sv = mb[b, pl.ds(j * L, L)]
            ty = mb[b, pl.ds(C + j * L, L)]
            dv = mb[b, pl.ds(2 * C + j * L, L)]
            tn = ty * N
            idxb[3 * b, pl.ds(j * L, L)] = tn + sv
            idxb[3 * b + 1, pl.ds(j * L, L)] = tn + dv + coff
            idxb[3 * b + 2, pl.ds(j * L, L)] = dv

    def fire_gathers(b):
        pltpu.async_copy(xr_hbm.at[idxb.at[3 * b]], rowsb.at[b], gsem.at[b])
        pltpu.async_copy(inv_hbm.at[idxb.at[3 * b + 1]], scaleb.at[b],
                         gsem.at[b])

    def wait_gathers(b):
        pltpu.make_async_copy(xr_hbm.at[idxb.at[3 * b]], rowsb.at[b],
                              gsem.at[b]).wait()
        pltpu.make_async_copy(inv_hbm.at[idxb.at[3 * b + 1]], scaleb.at[b],
                              gsem.at[b]).wait()

    def wait_scatter(b):
        pltpu.make_async_copy(rowsb.at[b], acc.at[idxb.at[3 * b + 2]],
                              ssem.at[b]).wait()

    for k0 in range(NBM):
        fire_meta(k0)
    wait_meta(0)
    compute_idx(0)
    fire_gathers(0)

    def step(k, carry):
        b = lax.rem(k, NBM)
        kn = k + 1
        bn = lax.rem(kn, NBM)

        @pl.when(kn < nch)
        def _():
            wait_meta(bn)

            @pl.when(kn >= NBM)
            def _():
                wait_scatter(bn)

            compute_idx(bn)
            fire_gathers(bn)

        @pl.when(k + NBM < nch)
        def _():
            fire_meta(k + NBM)

        wait_gathers(b)

        @plsc.parallel_loop(0, C // L, 1, unroll=2)
        def _scale(j):
            sv16 = scaleb[b, pl.ds(j * L, L)]
            for t in range(L):
                e = j * L + t
                sc = sv16[t]
                for kk in range(D // L):
                    rowsb[b, e, pl.ds(kk * L, L)] = (
                        rowsb[b, e, pl.ds(kk * L, L)] * sc)

        pltpu.async_copy(rowsb.at[b], acc.at[idxb.at[3 * b + 2]],
                         ssem.at[b], add=True)
        return carry

    lax.fori_loop(0, nch, step, 0)
    for b in range(NBM):
        wait_scatter(b)
    plsc.subcore_barrier()
    pltpu.sync_copy(acc.at[pl.ds(s * RPT, RPT)],
                    msg_hbm.at[c, pl.ds(s * RPT, RPT)])


_sc_main = pl.kernel(
    _sc_main_body,
    mesh=_MESH,
    out_type=[jax.ShapeDtypeStruct((NC, NP, D), jnp.float32),
              jax.ShapeDtypeStruct((NC * CNTP,), jnp.float32)],
    scratch_types=[
        pltpu.VMEM((NBM, RSZ), jnp.int32),
        pltpu.VMEM((3 * NBM, C), jnp.int32),
        pltpu.VMEM((NBM + 1, C), jnp.float32),
        pltpu.VMEM((NBM, C, D), jnp.float32),
        pltpu.VMEM((CPT,), jnp.float32),
        pltpu.VMEM((PIECE,), jnp.float32),
        pltpu.VMEM_SHARED((NP, D), jnp.float32),
        pltpu.SemaphoreType.DMA((NBM,)),
        pltpu.SemaphoreType.DMA((NBM,)),
        pltpu.SemaphoreType.DMA((NBM,)),
    ],
)


BN = 400


def _mm_body(x_ref, w_ref, bias_ref, o_ref):
    o_ref[0] = jnp.dot(x_ref[...], w_ref[0], preferred_element_type=jnp.float32)

    @pl.when(pl.program_id(0) == 8)
    def _():
        o_ref[0] += bias_ref[...]


_mm = pl.pallas_call(
    _mm_body,
    grid=(9, N // BN),
    in_specs=[pl.BlockSpec((BN, D), lambda r, i: (i, 0)),
              pl.BlockSpec((1, D, D), lambda r, i: (r, 0, 0)),
              pl.BlockSpec((1, D), lambda r, i: (0, 0))],
    out_specs=pl.BlockSpec((1, BN, D), lambda r, i: (r, i, 0)),
    out_shape=jax.ShapeDtypeStruct((9, N, D), jnp.float32),
)


def _fin_body(m_ref, o_ref):
    o_ref[...] = m_ref[0] + m_ref[1]


_fin = pl.pallas_call(
    _fin_body,
    grid=(N // BN,),
    in_specs=[pl.BlockSpec((2, BN, D), lambda i: (0, i, 0))],
    out_specs=pl.BlockSpec((BN, D), lambda i: (i, 0)),
    out_shape=jax.ShapeDtypeStruct((N, D), jnp.float32),
)


def kernel(x, edge_index, edge_type, rel_weight, self_loop_weight, bias):
    src = edge_index[0]
    dst = edge_index[1]
    meta = jnp.stack([src.reshape(G, C), edge_type.reshape(G, C),
                      dst.reshape(G, C)], axis=1).reshape(-1)
    w9 = jnp.concatenate([rel_weight, self_loop_weight[None]], axis=0)
    y9 = _mm(x, w9, bias.reshape(1, D))
    zeros1 = jnp.zeros((CNTP,), jnp.float32)
    zeros2 = jnp.zeros((NP, D), jnp.float32)
    base = jnp.concatenate([y9[8], jnp.zeros((NP - N, D), jnp.float32)],
                           axis=0)
    counts = _sc_count(meta, zeros1)
    msg, _ = _sc_main(meta, y9.reshape(9 * N, D), counts, base, zeros2)
    out = _fin(msg)
    return out


# speedup vs baseline: 1.1810x; 1.1810x over previous
import jax
import jax.numpy as jnp
from jax import lax
from jax.experimental import pallas as pl
from jax.experimental.pallas import tpu as pltpu
from jax.experimental.pallas import tpu_sc as plsc

N = 10000
E = 320000
D = 128
R = 8
RN = R * N
NC = 2
NS = 16
L = 16
NW = NC * NS
C = 128
G = E // C
GBASE = G // NW
GEXTRA = G - GBASE * NW
CBASE = G // NS
CEXTRA = G - CBASE * NS
RSZ = 3 * C
NB = 3
NBM = 2
NP = 10240
RPT = NP // NS
CPT = 5120
CNTP = NS * CPT
PIECE = 640

_MESH = plsc.VectorSubcoreMesh(core_axis_name="c", subcore_axis_name="s")


def _sc_count_body(meta_hbm, zeros_hbm, counts_hbm,
                   mb, idxb, onesb, cacc, msem, ssem):
    c = lax.axis_index("c")
    s = lax.axis_index("s")
    wid = c * NS + s
    nch = GBASE + jnp.where(wid < GEXTRA, 1, 0)
    pltpu.sync_copy(zeros_hbm.at[pl.ds(s * CPT, CPT)],
                    cacc.at[pl.ds(s * CPT, CPT)])
    for j in range(C // L):
        onesb[pl.ds(j * L, L)] = jnp.full((L,), 1.0, jnp.float32)
    plsc.subcore_barrier()

    def fire_meta(k):
        g = k * NW + wid
        b = k % NB
        pltpu.async_copy(meta_hbm.at[pl.ds(g * RSZ, RSZ)], mb.at[b],
                         msem.at[b])

    def wait_meta(b):
        pltpu.make_async_copy(meta_hbm.at[pl.ds(0, RSZ)], mb.at[b],
                              msem.at[b]).wait()

    fire_meta(0)
    fire_meta(1)

    def step(k, carry):
        b = lax.rem(k, NB)
        wait_meta(b)

        @pl.when(k + 2 < nch)
        def _():
            fire_meta(k + 2)

        @pl.when(k >= NB)
        def _():
            pltpu.make_async_copy(onesb, cacc.at[idxb.at[b]],
                                  ssem.at[b]).wait()

        for j in range(C // L):
            ty = mb[b, pl.ds(C + j * L, L)]
            dv = mb[b, pl.ds(2 * C + j * L, L)]
            idxb[b, pl.ds(j * L, L)] = ty * N + dv
        pltpu.async_copy(onesb, cacc.at[idxb.at[b]], ssem.at[b], add=True)
        return carry

    lax.fori_loop(0, nch, step, 0)
    for b in range(NB):
        pltpu.make_async_copy(onesb, cacc.at[idxb.at[b]], ssem.at[b]).wait()
    plsc.subcore_barrier()
    pltpu.sync_copy(cacc.at[pl.ds(s * CPT, CPT)],
                    counts_hbm.at[pl.ds(c * CNTP + s * CPT, CPT)])


_sc_count = pl.kernel(
    _sc_count_body,
    mesh=_MESH,
    out_type=jax.ShapeDtypeStruct((NC * CNTP,), jnp.float32),
    scratch_types=[
        pltpu.VMEM((NB, RSZ), jnp.int32),
        pltpu.VMEM((NB, C), jnp.int32),
        pltpu.VMEM((C,), jnp.float32),
        pltpu.VMEM_SHARED((CNTP,), jnp.float32),
        pltpu.SemaphoreType.DMA((NB,)),
        pltpu.SemaphoreType.DMA((NB,)),
    ],
)


def _sc_main_body(meta_hbm, xr_hbm, counts_hbm, base_hbm,
                  msg_hbm, inv_hbm, mb, idxb, scaleb, rowsb, cbuf, cbuf2,
                  acc, msem, gsem, ssem):
    c = lax.axis_index("c")
    s = lax.axis_index("s")
    wid = c * NS + s
    nch = GBASE + jnp.where(wid < GEXTRA, 1, 0)

    @pl.when(c == 0)
    def _():
        pltpu.sync_copy(base_hbm.at[pl.ds(s * RPT, RPT)],
                        acc.at[pl.ds(s * RPT, RPT)])

    @pl.when(c == 1)
    def _():
        @plsc.parallel_loop(0, C, 1, unroll=4)
        def _z(e):
            for kk in range(D // L):
                rowsb[0, e, pl.ds(kk * L, L)] = jnp.zeros((L,), jnp.float32)

        for q in range(RPT // C):
            pltpu.sync_copy(rowsb.at[0],
                            acc.at[pl.ds(s * RPT + q * C, C)])

    pltpu.sync_copy(counts_hbm.at[pl.ds(c * CNTP + s * CPT, CPT)], cbuf)
    oc = 1 - c
    for p in range(CPT // PIECE):
        pltpu.sync_copy(
            counts_hbm.at[pl.ds(oc * CNTP + s * CPT + p * PIECE, PIECE)],
            cbuf2)

        @plsc.parallel_loop(0, PIECE // L, 1, unroll=4)
        def _inv(i):
            v = cbuf[pl.ds(p * PIECE + i * L, L)] + cbuf2[pl.ds(i * L, L)]
            cbuf[pl.ds(p * PIECE + i * L, L)] = 1.0 / jnp.maximum(v, 1.0)

    pltpu.sync_copy(cbuf, inv_hbm.at[pl.ds(c * CNTP + s * CPT, CPT)])
    plsc.subcore_barrier()

    def fire_meta(k):
        g = k * NW + wid
        b = k % NBM
        pltpu.async_copy(meta_hbm.at[pl.ds(g * RSZ, RSZ)], mb.at[b],
                         msem.at[b])

    def wait_meta(b):
        pltpu.make_async_copy(meta_hbm.at[pl.ds(0, RSZ)], mb.at[b],
                              msem.at[b]).wait()

    coff = c * CNTP

    def compute_idx(b):
        for j in range(C // L):
            sv = mb[b, pl.ds(j * L, L)]
            ty = mb[b, pl.ds(C + j * L, L)]
            dv = mb[b, pl.ds(2 * C + j * L, L)]
            idxb[3 * b, pl.ds(j * L, L)] = sv * (R + 1) + ty
            idxb[3 * b + 1, pl.ds(j * L, L)] = ty * N + dv + coff
            idxb[3 * b + 2, pl.ds(j * L, L)] = dv

    def fire_gathers(b):
        pltpu.async_copy(xr_hbm.at[idxb.at[3 * b]], rowsb.at[b], gsem.at[b])
        pltpu.async_copy(inv_hbm.at[idxb.at[3 * b + 1]], scaleb.at[b],
                         gsem.at[b])

    def wait_gathers(b):
        pltpu.make_async_copy(xr_hbm.at[idxb.at[3 * b]], rowsb.at[b],
                              gsem.at[b]).wait()
        pltpu.make_async_copy(inv_hbm.at[idxb.at[3 * b + 1]], scaleb.at[b],
                              gsem.at[b]).wait()

    def wait_scatter(b):
        pltpu.make_async_copy(rowsb.at[b], acc.at[idxb.at[3 * b + 2]],
                              ssem.at[b]).wait()

    for k0 in range(NBM):
        fire_meta(k0)
    wait_meta(0)
    compute_idx(0)
    fire_gathers(0)

    def step(k, carry):
        b = lax.rem(k, NBM)
        kn = k + 1
        bn = lax.rem(kn, NBM)

        @pl.when(kn < nch)
        def _():
            wait_meta(bn)

            @pl.when(kn >= NBM)
            def _():
                wait_scatter(bn)

            compute_idx(bn)
            fire_gathers(bn)

        @pl.when(k + NBM < nch)
        def _():
            fire_meta(k + NBM)

        wait_gathers(b)

        @plsc.parallel_loop(0, C // L, 1, unroll=2)
        def _scale(j):
            sv16 = scaleb[b, pl.ds(j * L, L)]
            for t in range(L):
                e = j * L + t
                sc = sv16[t]
                for kk in range(D // L):
                    rowsb[b, e, pl.ds(kk * L, L)] = (
                        rowsb[b, e, pl.ds(kk * L, L)] * sc)

        pltpu.async_copy(rowsb.at[b], acc.at[idxb.at[3 * b + 2]],
                         ssem.at[b], add=True)
        return carry

    lax.fori_loop(0, nch, step, 0)
    for b in range(NBM):
        wait_scatter(b)
    plsc.subcore_barrier()
    pltpu.sync_copy(acc.at[pl.ds(s * RPT, RPT)],
                    msg_hbm.at[c, pl.ds(s * RPT, RPT)])


_sc_main = pl.kernel(
    _sc_main_body,
    mesh=_MESH,
    out_type=[jax.ShapeDtypeStruct((NC, NP, D), jnp.float32),
              jax.ShapeDtypeStruct((NC * CNTP,), jnp.float32)],
    scratch_types=[
        pltpu.VMEM((NBM, RSZ), jnp.int32),
        pltpu.VMEM((3 * NBM, C), jnp.int32),
        pltpu.VMEM((NBM + 1, C), jnp.float32),
        pltpu.VMEM((NBM, C, D), jnp.float32),
        pltpu.VMEM((CPT,), jnp.float32),
        pltpu.VMEM((PIECE,), jnp.float32),
        pltpu.VMEM_SHARED((NP, D), jnp.float32),
        pltpu.SemaphoreType.DMA((NBM,)),
        pltpu.SemaphoreType.DMA((NBM,)),
        pltpu.SemaphoreType.DMA((NBM,)),
    ],
)


BN = 400
BNN = 2000
DCAT = (R + 1) * D


def _mm_body(x_ref, w_ref, bias_ref, o_ref):
    o_ref[...] = jnp.dot(x_ref[...], w_ref[...],
                         preferred_element_type=jnp.float32) + bias_ref[...]


_mm = pl.pallas_call(
    _mm_body,
    grid=(N // BNN,),
    in_specs=[pl.BlockSpec((BNN, D), lambda i: (i, 0)),
              pl.BlockSpec((D, DCAT), lambda i: (0, 0)),
              pl.BlockSpec((1, DCAT), lambda i: (0, 0))],
    out_specs=pl.BlockSpec((BNN, DCAT), lambda i: (i, 0)),
    out_shape=jax.ShapeDtypeStruct((N, DCAT), jnp.float32),
)


def _fin_body(m_ref, o_ref):
    o_ref[...] = m_ref[0] + m_ref[1]


_fin = pl.pallas_call(
    _fin_body,
    grid=(N // BN,),
    in_specs=[pl.BlockSpec((2, BN, D), lambda i: (0, i, 0))],
    out_specs=pl.BlockSpec((BN, D), lambda i: (i, 0)),
    out_shape=jax.ShapeDtypeStruct((N, D), jnp.float32),
)


def kernel(x, edge_index, edge_type, rel_weight, self_loop_weight, bias):
    src = edge_index[0]
    dst = edge_index[1]
    meta = jnp.stack([src.reshape(G, C), edge_type.reshape(G, C),
                      dst.reshape(G, C)], axis=1).reshape(-1)
    wcat = jnp.concatenate(
        [rel_weight.transpose(1, 0, 2).reshape(D, R * D),
         self_loop_weight], axis=1)
    bias9 = jnp.concatenate(
        [jnp.zeros((R * D,), jnp.float32), bias]).reshape(1, DCAT)
    ycat = _mm(x, wcat, bias9)
    zeros1 = jnp.zeros((CNTP,), jnp.float32)
    base = jnp.concatenate([ycat[:, R * D:],
                            jnp.zeros((NP - N, D), jnp.float32)],
                           axis=0)
    counts = _sc_count(meta, zeros1)
    msg, _ = _sc_main(meta, ycat.reshape((R + 1) * N, D), counts, base)
    out = _fin(msg)
    return out


# speedup vs baseline: 1.1812x; 1.0002x over previous
import jax
import jax.numpy as jnp
from jax import lax
from jax.experimental import pallas as pl
from jax.experimental.pallas import tpu as pltpu
from jax.experimental.pallas import tpu_sc as plsc

N = 10000
E = 320000
D = 128
R = 8
RN = R * N
NC = 2
NS = 16
L = 16
NW = NC * NS
C = 128
G = E // C
GBASE = G // NW
GEXTRA = G - GBASE * NW
CBASE = G // NS
CEXTRA = G - CBASE * NS
RSZ = 3 * C
NB = 3
NBM = 2
NP = 10240
RPT = NP // NS
CPT = 5120
CNTP = NS * CPT
PIECE = 640

_MESH = plsc.VectorSubcoreMesh(core_axis_name="c", subcore_axis_name="s")


def _sc_count_body(meta_hbm, zeros_hbm, counts_hbm,
                   mb, idxb, onesb, cacc, msem, ssem):
    c = lax.axis_index("c")
    s = lax.axis_index("s")
    wid = c * NS + s
    nch = GBASE + jnp.where(wid < GEXTRA, 1, 0)
    pltpu.sync_copy(zeros_hbm.at[pl.ds(s * CPT, CPT)],
                    cacc.at[pl.ds(s * CPT, CPT)])
    for j in range(C // L):
        onesb[pl.ds(j * L, L)] = jnp.full((L,), 1.0, jnp.float32)
    plsc.subcore_barrier()

    def fire_meta(k):
        g = k * NW + wid
        b = k % NB
        pltpu.async_copy(meta_hbm.at[pl.ds(g * RSZ, RSZ)], mb.at[b],
                         msem.at[b])

    def wait_meta(b):
        pltpu.make_async_copy(meta_hbm.at[pl.ds(0, RSZ)], mb.at[b],
                              msem.at[b]).wait()

    fire_meta(0)
    fire_meta(1)

    def step(k, carry):
        b = lax.rem(k, NB)
        wait_meta(b)

        @pl.when(k + 2 < nch)
        def _():
            fire_meta(k + 2)

        @pl.when(k >= NB)
        def _():
            pltpu.make_async_copy(onesb, cacc.at[idxb.at[b]],
                                  ssem.at[b]).wait()

        for j in range(C // L):
            ty = mb[b, pl.ds(C + j * L, L)]
            dv = mb[b, pl.ds(2 * C + j * L, L)]
            idxb[b, pl.ds(j * L, L)] = ty * N + dv
        pltpu.async_copy(onesb, cacc.at[idxb.at[b]], ssem.at[b], add=True)
        return carry

    lax.fori_loop(0, nch, step, 0)
    for b in range(NB):
        pltpu.make_async_copy(onesb, cacc.at[idxb.at[b]], ssem.at[b]).wait()
    plsc.subcore_barrier()
    pltpu.sync_copy(cacc.at[pl.ds(s * CPT, CPT)],
                    counts_hbm.at[pl.ds(c * CNTP + s * CPT, CPT)])


_sc_count = pl.kernel(
    _sc_count_body,
    mesh=_MESH,
    out_type=jax.ShapeDtypeStruct((NC * CNTP,), jnp.float32),
    scratch_types=[
        pltpu.VMEM((NB, RSZ), jnp.int32),
        pltpu.VMEM((NB, C), jnp.int32),
        pltpu.VMEM((C,), jnp.float32),
        pltpu.VMEM_SHARED((CNTP,), jnp.float32),
        pltpu.SemaphoreType.DMA((NB,)),
        pltpu.SemaphoreType.DMA((NB,)),
    ],
)


def _sc_main_body(meta_hbm, xr_hbm, counts_hbm, base_hbm,
                  msg_hbm, inv_hbm, mb, idxb, scaleb, rowsb, cbuf, cbuf2,
                  acc, msem, gsem, ssem):
    c = lax.axis_index("c")
    s = lax.axis_index("s")
    wid = c * NS + s
    nch = GBASE + jnp.where(wid < GEXTRA, 1, 0)

    @pl.when(c == 0)
    def _():
        pltpu.sync_copy(base_hbm.at[pl.ds(s * RPT, RPT)],
                        acc.at[pl.ds(s * RPT, RPT)])

    @pl.when(c == 1)
    def _():
        @plsc.parallel_loop(0, C, 1, unroll=4)
        def _z(e):
            for kk in range(D // L):
                rowsb[0, e, pl.ds(kk * L, L)] = jnp.zeros((L,), jnp.float32)

        for q in range(RPT // C):
            pltpu.sync_copy(rowsb.at[0],
                            acc.at[pl.ds(s * RPT + q * C, C)])

    pltpu.sync_copy(counts_hbm.at[pl.ds(c * CNTP + s * CPT, CPT)], cbuf)
    oc = 1 - c
    for p in range(CPT // PIECE):
        pltpu.sync_copy(
            counts_hbm.at[pl.ds(oc * CNTP + s * CPT + p * PIECE, PIECE)],
            cbuf2)

        @plsc.parallel_loop(0, PIECE // L, 1, unroll=4)
        def _inv(i):
            v = cbuf[pl.ds(p * PIECE + i * L, L)] + cbuf2[pl.ds(i * L, L)]
            cbuf[pl.ds(p * PIECE + i * L, L)] = 1.0 / jnp.maximum(v, 1.0)

    pltpu.sync_copy(cbuf, inv_hbm.at[pl.ds(c * CNTP + s * CPT, CPT)])
    plsc.subcore_barrier()

    def fire_meta(k):
        g = k * NW + wid
        b = k % NBM
        pltpu.async_copy(meta_hbm.at[pl.ds(g * RSZ, RSZ)], mb.at[b],
                         msem.at[b])

    def wait_meta(b):
        pltpu.make_async_copy(meta_hbm.at[pl.ds(0, RSZ)], mb.at[b],
                              msem.at[b]).wait()

    coff = c * CNTP

    def compute_idx(b):
        for j in range(C // L):
            sv = mb[b, pl.ds(j * L, L)]
            ty = mb[b, pl.ds(C + j * L, L)]
            dv = mb[b, pl.ds(2 * C + j * L, L)]
            idxb[3 * b, pl.ds(j * L, L)] = sv * (R + 1) + ty
            idxb[3 * b + 1, pl.ds(j * L, L)] = ty * N + dv + coff
            idxb[3 * b + 2, pl.ds(j * L, L)] = dv

    def fire_gathers(b):
        pltpu.async_copy(xr_hbm.at[idxb.at[3 * b]], rowsb.at[b], gsem.at[b])
        pltpu.async_copy(inv_hbm.at[idxb.at[3 * b + 1]], scaleb.at[b],
                         gsem.at[b])

    def wait_gathers(b):
        pltpu.make_async_copy(xr_hbm.at[idxb.at[3 * b]], rowsb.at[b],
                              gsem.at[b]).wait()
        pltpu.make_async_copy(inv_hbm.at[idxb.at[3 * b + 1]], scaleb.at[b],
                              gsem.at[b]).wait()

    def wait_scatter(b):
        pltpu.make_async_copy(rowsb.at[b], acc.at[idxb.at[3 * b + 2]],
                              ssem.at[b]).wait()

    for k0 in range(NBM):
        fire_meta(k0)
    wait_meta(0)
    compute_idx(0)
    fire_gathers(0)

    def step(k, carry):
        b = lax.rem(k, NBM)
        kn = k + 1
        bn = lax.rem(kn, NBM)

        @pl.when(kn < nch)
        def _():
            wait_meta(bn)

            @pl.when(kn >= NBM)
            def _():
                wait_scatter(bn)

            compute_idx(bn)
            fire_gathers(bn)

        @pl.when(k + NBM < nch)
        def _():
            fire_meta(k + NBM)

        wait_gathers(b)

        @plsc.parallel_loop(0, C // L, 1, unroll=2)
        def _scale(j):
            sv16 = scaleb[b, pl.ds(j * L, L)]
            for t in range(L):
                e = j * L + t
                sv = sv16.at[jnp.full((L,), t, jnp.int32)].get(
                    mode="promise_in_bounds")
                for kk in range(D // L):
                    rowsb[b, e, pl.ds(kk * L, L)] = (
                        rowsb[b, e, pl.ds(kk * L, L)] * sv)

        pltpu.async_copy(rowsb.at[b], acc.at[idxb.at[3 * b + 2]],
                         ssem.at[b], add=True)
        return carry

    lax.fori_loop(0, nch, step, 0)
    for b in range(NBM):
        wait_scatter(b)
    plsc.subcore_barrier()
    pltpu.sync_copy(acc.at[pl.ds(s * RPT, RPT)],
                    msg_hbm.at[c, pl.ds(s * RPT, RPT)])


_sc_main = pl.kernel(
    _sc_main_body,
    mesh=_MESH,
    out_type=[jax.ShapeDtypeStruct((NC, NP, D), jnp.float32),
              jax.ShapeDtypeStruct((NC * CNTP,), jnp.float32)],
    scratch_types=[
        pltpu.VMEM((NBM, RSZ), jnp.int32),
        pltpu.VMEM((3 * NBM, C), jnp.int32),
        pltpu.VMEM((NBM + 1, C), jnp.float32),
        pltpu.VMEM((NBM, C, D), jnp.float32),
        pltpu.VMEM((CPT,), jnp.float32),
        pltpu.VMEM((PIECE,), jnp.float32),
        pltpu.VMEM_SHARED((NP, D), jnp.float32),
        pltpu.SemaphoreType.DMA((NBM,)),
        pltpu.SemaphoreType.DMA((NBM,)),
        pltpu.SemaphoreType.DMA((NBM,)),
    ],
)


BN = 400
BNN = 2000
DCAT = (R + 1) * D


def _mm_body(x_ref, w_ref, bias_ref, o_ref):
    o_ref[...] = jnp.dot(x_ref[...], w_ref[...],
                         preferred_element_type=jnp.float32) + bias_ref[...]


_mm = pl.pallas_call(
    _mm_body,
    grid=(N // BNN,),
    in_specs=[pl.BlockSpec((BNN, D), lambda i: (i, 0)),
              pl.BlockSpec((D, DCAT), lambda i: (0, 0)),
              pl.BlockSpec((1, DCAT), lambda i: (0, 0))],
    out_specs=pl.BlockSpec((BNN, DCAT), lambda i: (i, 0)),
    out_shape=jax.ShapeDtypeStruct((N, DCAT), jnp.float32),
)


def _fin_body(m_ref, o_ref):
    o_ref[...] = m_ref[0] + m_ref[1]


_fin = pl.pallas_call(
    _fin_body,
    grid=(N // BN,),
    in_specs=[pl.BlockSpec((2, BN, D), lambda i: (0, i, 0))],
    out_specs=pl.BlockSpec((BN, D), lambda i: (i, 0)),
    out_shape=jax.ShapeDtypeStruct((N, D), jnp.float32),
)


def kernel(x, edge_index, edge_type, rel_weight, self_loop_weight, bias):
    src = edge_index[0]
    dst = edge_index[1]
    meta = jnp.stack([src.reshape(G, C), edge_type.reshape(G, C),
                      dst.reshape(G, C)], axis=1).reshape(-1)
    wcat = jnp.concatenate(
        [rel_weight.transpose(1, 0, 2).reshape(D, R * D),
         self_loop_weight], axis=1)
    bias9 = jnp.concatenate(
        [jnp.zeros((R * D,), jnp.float32), bias]).reshape(1, DCAT)
    ycat = _mm(x, wcat, bias9)
    zeros1 = jnp.zeros((CNTP,), jnp.float32)
    base = jnp.concatenate([ycat[:, R * D:],
                            jnp.zeros((NP - N, D), jnp.float32)],
                           axis=0)
    counts = _sc_count(meta, zeros1)
    msg, _ = _sc_main(meta, ycat.reshape((R + 1) * N, D), counts, base)
    out = _fin(msg)
    return out
